# baseline (speedup 1.0000x reference)
import functools

import jax
import jax.numpy as jnp
from jax import lax
from jax.experimental import pallas as pl
from jax.experimental.pallas import tpu as pltpu
from jax.experimental.pallas import tpu_sc as plsc

_ALPHA = 0.8
_N_ROWS = 16384
_N_COLS = 128
_N_IDX = 8192

_ROW_BLK = 2048
_GRID = _N_ROWS // _ROW_BLK

_NC = 2
_NS = 16
_NW = _NC * _NS
_IDX_PER_TILE = _N_IDX // _NW
_LANES = 16


def _rowsum_body(inp_ref, tgt_ref, out_ref):
    d = inp_ref[...] - tgt_ref[...]
    ones = jnp.ones((_N_COLS, 1), jnp.float32)
    out_ref[...] = jax.lax.dot_general(
        d * d, ones, (((1,), (0,)), ((), ())),
        preferred_element_type=jnp.float32)


_rowsum_call = pl.pallas_call(
    _rowsum_body,
    grid=(_GRID,),
    in_specs=[
        pl.BlockSpec((_ROW_BLK, _N_COLS), lambda i: (i, 0)),
        pl.BlockSpec((_ROW_BLK, _N_COLS), lambda i: (i, 0)),
    ],
    out_specs=pl.BlockSpec((_ROW_BLK, 1), lambda i: (i, 0)),
    out_shape=jax.ShapeDtypeStruct((_N_ROWS, 1), jnp.float32),
)


def _sc_gather_body(rowsum_hbm, one_hbm, zero_hbm, out_hbm,
                    table_v, one_v, zero_v, out_v):
    cid = lax.axis_index("c")
    sid = lax.axis_index("s")
    wid = sid * _NC + cid
    base = wid * _IDX_PER_TILE

    pltpu.sync_copy(rowsum_hbm, table_v)
    pltpu.sync_copy(one_hbm.at[pl.ds(base, _IDX_PER_TILE)], one_v)
    pltpu.sync_copy(zero_hbm.at[pl.ds(base, _IDX_PER_TILE)], zero_v)

    acc1 = jnp.zeros((_LANES,), jnp.float32)
    acc0 = jnp.zeros((_LANES,), jnp.float32)
    for i in range(_IDX_PER_TILE // _LANES):
        i1 = one_v[pl.ds(i * _LANES, _LANES)]
        i0 = zero_v[pl.ds(i * _LANES, _LANES)]
        acc1 = acc1 + plsc.load_gather(table_v, [i1])
        acc0 = acc0 + plsc.load_gather(table_v, [i0])
    acc = jnp.float32(1.0 - _ALPHA) * acc1 + jnp.float32(_ALPHA) * acc0
    total = jnp.sum(acc)
    out_v[...] = jnp.full((_LANES,), total, jnp.float32)
    pltpu.sync_copy(out_v, out_hbm.at[wid])


_sc_gather_call = functools.partial(
    pl.kernel,
    mesh=plsc.VectorSubcoreMesh(core_axis_name="c", subcore_axis_name="s"),
    out_type=jax.ShapeDtypeStruct((_NW, _LANES), jnp.float32),
    scratch_types=[
        pltpu.VMEM((_N_ROWS,), jnp.float32),
        pltpu.VMEM((_IDX_PER_TILE,), jnp.int32),
        pltpu.VMEM((_IDX_PER_TILE,), jnp.int32),
        pltpu.VMEM((_LANES,), jnp.float32),
    ],
    compiler_params=pltpu.CompilerParams(needs_layout_passes=False),
)(_sc_gather_body)


def kernel(one_index, zero_index, target, input):
    rowsum = _rowsum_call(input, target).reshape(_N_ROWS)
    partials = _sc_gather_call(rowsum, one_index, zero_index)
    return jnp.sum(partials[:, 0])


# speedup vs baseline: 1.0911x; 1.0911x over previous
import functools

import jax
import jax.numpy as jnp
from jax import lax
from jax.experimental import pallas as pl
from jax.experimental.pallas import tpu as pltpu
from jax.experimental.pallas import tpu_sc as plsc

_ALPHA = 0.8
_N_ROWS = 16384
_N_COLS = 128
_N_IDX = 8192

_ROW_BLK = 2048
_GRID = _N_ROWS // _ROW_BLK

_NC = 2
_NS = 16
_NW = _NC * _NS
_ROWS_PER_TILE = _N_ROWS // _NW
_LANES = 16
_UNROLL = 8


def _rowsum_body(inp_ref, tgt_ref, out_ref):
    d = inp_ref[...] - tgt_ref[...]
    ones = jnp.ones((_N_COLS, 1), jnp.float32)
    out_ref[...] = jax.lax.dot_general(
        d * d, ones, (((1,), (0,)), ((), ())),
        preferred_element_type=jnp.float32)


_rowsum_call = pl.pallas_call(
    _rowsum_body,
    grid=(_GRID,),
    in_specs=[
        pl.BlockSpec((_ROW_BLK, _N_COLS), lambda i: (i, 0)),
        pl.BlockSpec((_ROW_BLK, _N_COLS), lambda i: (i, 0)),
    ],
    out_specs=pl.BlockSpec((_ROW_BLK, 1), lambda i: (i, 0)),
    out_shape=jax.ShapeDtypeStruct((_N_ROWS, 1), jnp.float32),
)


def _sc_hist_body(one_hbm, zero_hbm, w_hbm, one_v, zero_v, w_v):
    cid = lax.axis_index("c")
    sid = lax.axis_index("s")
    wid = sid * _NC + cid
    base = wid * _ROWS_PER_TILE

    pltpu.sync_copy(one_hbm, one_v)
    pltpu.sync_copy(zero_hbm, zero_v)

    zero_vec = jnp.zeros((_LANES,), jnp.float32)
    for i in range(_ROWS_PER_TILE // _LANES):
        w_v[pl.ds(i * _LANES, _LANES)] = zero_vec

    lo = jnp.int32(base)
    hi = jnp.int32(base + _ROWS_PER_TILE)

    def scan(idx_v, weight):
        wvec = jnp.full((_LANES,), weight, jnp.float32)

        def body(j, carry):
            for u in range(_UNROLL):
                v = idx_v[pl.ds((j * _UNROLL + u) * _LANES, _LANES)]
                mask = (v >= lo) & (v < hi)
                plsc.addupdate_scatter(w_v, [v - lo], wvec, mask=mask)
            return carry

        lax.fori_loop(0, _N_IDX // _LANES // _UNROLL, body, jnp.int32(0))

    scan(one_v, 1.0 - _ALPHA)
    scan(zero_v, _ALPHA)

    pltpu.sync_copy(w_v, w_hbm.at[pl.ds(base, _ROWS_PER_TILE)])


_sc_hist_call = functools.partial(
    pl.kernel,
    mesh=plsc.VectorSubcoreMesh(core_axis_name="c", subcore_axis_name="s"),
    out_type=jax.ShapeDtypeStruct((_N_ROWS,), jnp.float32),
    scratch_types=[
        pltpu.VMEM((_N_IDX,), jnp.int32),
        pltpu.VMEM((_N_IDX,), jnp.int32),
        pltpu.VMEM((_ROWS_PER_TILE,), jnp.float32),
    ],
    compiler_params=pltpu.CompilerParams(needs_layout_passes=False),
)(_sc_hist_body)


def _dot_body(w_ref, r_ref, out_ref):
    out_ref[...] = jnp.sum(w_ref[...] * r_ref[...]).reshape(1, 1)


_dot_call = pl.pallas_call(
    _dot_body,
    out_shape=jax.ShapeDtypeStruct((1, 1), jnp.float32),
)


def kernel(one_index, zero_index, target, input):
    weights = _sc_hist_call(one_index, zero_index)
    rowsum = _rowsum_call(input, target)
    loss = _dot_call(weights.reshape(_N_ROWS // _N_COLS, _N_COLS),
                     rowsum.reshape(_N_ROWS // _N_COLS, _N_COLS))
    return loss[0, 0]


# speedup vs baseline: 1.1090x; 1.0164x over previous
import functools

import jax
import jax.numpy as jnp
from jax import lax
from jax.experimental import pallas as pl
from jax.experimental.pallas import tpu as pltpu
from jax.experimental.pallas import tpu_sc as plsc

_ALPHA = 0.8
_N_ROWS = 16384
_N_COLS = 128
_N_IDX = 8192

_ROW_BLK = 2048
_GRID = _N_ROWS // _ROW_BLK

_NC = 2
_NS = 16
_NW = _NC * _NS
_LANES = 16
_IDX_BLK = 128
_IDX_ROWS = _N_IDX // _IDX_BLK
_ROWS_PER_TILE = _IDX_ROWS // _NW
_HIST_SLICE = _N_ROWS // _NS


def _rowsum_body(inp_ref, tgt_ref, out_ref):
    d = inp_ref[...] - tgt_ref[...]
    ones = jnp.ones((_N_COLS, 1), jnp.float32)
    out_ref[...] = jax.lax.dot_general(
        d * d, ones, (((1,), (0,)), ((), ())),
        preferred_element_type=jnp.float32)


_rowsum_call = pl.pallas_call(
    _rowsum_body,
    grid=(_GRID,),
    in_specs=[
        pl.BlockSpec((_ROW_BLK, _N_COLS), lambda i: (i, 0)),
        pl.BlockSpec((_ROW_BLK, _N_COLS), lambda i: (i, 0)),
    ],
    out_specs=pl.BlockSpec((_ROW_BLK, 1), lambda i: (i, 0)),
    out_shape=jax.ShapeDtypeStruct((_N_ROWS, 1), jnp.float32),
)


def _sc_hist_body(one_hbm, zero_hbm, w_hbm,
                  idx1_v, idx0_v, vals1_v, vals0_v, zeros_v, hist_s):
    cid = lax.axis_index("c")
    sid = lax.axis_index("s")

    pltpu.sync_copy(one_hbm.at[pl.ds(sid * _ROWS_PER_TILE * _NC
                                     + cid * _ROWS_PER_TILE, _ROWS_PER_TILE)],
                    idx1_v)
    pltpu.sync_copy(zero_hbm.at[pl.ds(sid * _ROWS_PER_TILE * _NC
                                      + cid * _ROWS_PER_TILE, _ROWS_PER_TILE)],
                    idx0_v)

    w1 = jnp.full((_LANES,), 1.0 - _ALPHA, jnp.float32)
    w0 = jnp.full((_LANES,), _ALPHA, jnp.float32)
    zv = jnp.zeros((_LANES,), jnp.float32)
    for i in range(_IDX_BLK // _LANES):
        vals1_v[pl.ds(i * _LANES, _LANES)] = w1
        vals0_v[pl.ds(i * _LANES, _LANES)] = w0
    for i in range(_HIST_SLICE // _LANES):
        zeros_v[pl.ds(i * _LANES, _LANES)] = zv

    pltpu.sync_copy(zeros_v, hist_s.at[pl.ds(sid * _HIST_SLICE, _HIST_SLICE)])
    plsc.subcore_barrier()

    for r in range(_ROWS_PER_TILE):
        pltpu.sync_copy(vals1_v, hist_s.at[idx1_v.at[r]], add=True)
        pltpu.sync_copy(vals0_v, hist_s.at[idx0_v.at[r]], add=True)
    plsc.subcore_barrier()

    pltpu.sync_copy(hist_s.at[pl.ds(sid * _HIST_SLICE, _HIST_SLICE)],
                    w_hbm.at[cid, pl.ds(sid * _HIST_SLICE, _HIST_SLICE)])


_sc_hist_call = functools.partial(
    pl.kernel,
    mesh=plsc.VectorSubcoreMesh(core_axis_name="c", subcore_axis_name="s"),
    out_type=jax.ShapeDtypeStruct((_NC, _N_ROWS), jnp.float32),
    scratch_types=[
        pltpu.VMEM((_ROWS_PER_TILE, _IDX_BLK), jnp.int32),
        pltpu.VMEM((_ROWS_PER_TILE, _IDX_BLK), jnp.int32),
        pltpu.VMEM((_IDX_BLK,), jnp.float32),
        pltpu.VMEM((_IDX_BLK,), jnp.float32),
        pltpu.VMEM((_HIST_SLICE,), jnp.float32),
        pltpu.VMEM_SHARED((_N_ROWS,), jnp.float32),
    ],
    compiler_params=pltpu.CompilerParams(needs_layout_passes=False),
)(_sc_hist_body)


_W_ROWS = _N_ROWS // _N_COLS


def _dot_body(w_ref, r_ref, out_ref):
    w = w_ref[:_W_ROWS, :] + w_ref[_W_ROWS:, :]
    out_ref[...] = jnp.sum(w * r_ref[...]).reshape(1, 1)


_dot_call = pl.pallas_call(
    _dot_body,
    out_shape=jax.ShapeDtypeStruct((1, 1), jnp.float32),
)


def kernel(one_index, zero_index, target, input):
    one2 = one_index.reshape(_IDX_ROWS, _IDX_BLK)
    zero2 = zero_index.reshape(_IDX_ROWS, _IDX_BLK)
    weights = _sc_hist_call(one2, zero2)
    rowsum = _rowsum_call(input, target)
    loss = _dot_call(weights.reshape(_NC * _W_ROWS, _N_COLS),
                     rowsum.reshape(_W_ROWS, _N_COLS))
    return loss[0, 0]


# speedup vs baseline: 1.4917x; 1.3451x over previous
import functools

import jax
import jax.numpy as jnp
from jax import lax
from jax.experimental import pallas as pl
from jax.experimental.pallas import tpu as pltpu
from jax.experimental.pallas import tpu_sc as plsc

_ALPHA = 0.8
_N_ROWS = 16384
_N_COLS = 128
_N_IDX = 8192

_ROW_BLK = 2048
_GRID = _N_ROWS // _ROW_BLK

_NC = 2
_NS = 16
_NW = _NC * _NS
_LANES = 16
_IDX_BLK = 128
_IDX_ROWS = _N_IDX // _IDX_BLK
_ROWS_PER_TILE = _IDX_ROWS // _NW
_HIST_SLICE = _N_ROWS // _NS


_W_ROWS = _N_ROWS // _N_COLS
_SUB = _ROW_BLK // _N_COLS


def _rowsum_body(inp_ref, tgt_ref, out_ref):
    ones = jnp.ones((1, _N_COLS), jnp.float32)
    for k in range(_SUB):
        d = (inp_ref[pl.ds(k * _N_COLS, _N_COLS), :]
             - tgt_ref[pl.ds(k * _N_COLS, _N_COLS), :])
        out_ref[pl.ds(k, 1), :] = jax.lax.dot_general(
            ones, d * d, (((1,), (1,)), ((), ())),
            preferred_element_type=jnp.float32)


_rowsum_call = pl.pallas_call(
    _rowsum_body,
    grid=(_GRID,),
    in_specs=[
        pl.BlockSpec((_ROW_BLK, _N_COLS), lambda i: (i, 0)),
        pl.BlockSpec((_ROW_BLK, _N_COLS), lambda i: (i, 0)),
    ],
    out_specs=pl.BlockSpec((_SUB, _N_COLS), lambda i: (i, 0)),
    out_shape=jax.ShapeDtypeStruct((_W_ROWS, _N_COLS), jnp.float32),
)


def _sc_hist_body(one_hbm, zero_hbm, w_hbm,
                  idx1_v, idx0_v, vals1_v, vals0_v, zeros_v, hist_s):
    cid = lax.axis_index("c")
    sid = lax.axis_index("s")

    pltpu.sync_copy(one_hbm.at[pl.ds(sid * _ROWS_PER_TILE * _NC
                                     + cid * _ROWS_PER_TILE, _ROWS_PER_TILE)],
                    idx1_v)
    pltpu.sync_copy(zero_hbm.at[pl.ds(sid * _ROWS_PER_TILE * _NC
                                      + cid * _ROWS_PER_TILE, _ROWS_PER_TILE)],
                    idx0_v)

    w1 = jnp.full((_LANES,), 1.0 - _ALPHA, jnp.float32)
    w0 = jnp.full((_LANES,), _ALPHA, jnp.float32)
    zv = jnp.zeros((_LANES,), jnp.float32)
    for i in range(_IDX_BLK // _LANES):
        vals1_v[pl.ds(i * _LANES, _LANES)] = w1
        vals0_v[pl.ds(i * _LANES, _LANES)] = w0
    for i in range(_HIST_SLICE // _LANES):
        zeros_v[pl.ds(i * _LANES, _LANES)] = zv

    pltpu.sync_copy(zeros_v, hist_s.at[pl.ds(sid * _HIST_SLICE, _HIST_SLICE)])
    plsc.subcore_barrier()

    for r in range(_ROWS_PER_TILE):
        pltpu.sync_copy(vals1_v, hist_s.at[idx1_v.at[r]], add=True)
        pltpu.sync_copy(vals0_v, hist_s.at[idx0_v.at[r]], add=True)
    plsc.subcore_barrier()

    pltpu.sync_copy(hist_s.at[pl.ds(sid * _HIST_SLICE, _HIST_SLICE)],
                    w_hbm.at[cid, pl.ds(sid * _HIST_SLICE, _HIST_SLICE)])


_sc_hist_call = functools.partial(
    pl.kernel,
    mesh=plsc.VectorSubcoreMesh(core_axis_name="c", subcore_axis_name="s"),
    out_type=jax.ShapeDtypeStruct((_NC, _N_ROWS), jnp.float32),
    scratch_types=[
        pltpu.VMEM((_ROWS_PER_TILE, _IDX_BLK), jnp.int32),
        pltpu.VMEM((_ROWS_PER_TILE, _IDX_BLK), jnp.int32),
        pltpu.VMEM((_IDX_BLK,), jnp.float32),
        pltpu.VMEM((_IDX_BLK,), jnp.float32),
        pltpu.VMEM((_HIST_SLICE,), jnp.float32),
        pltpu.VMEM_SHARED((_N_ROWS,), jnp.float32),
    ],
    compiler_params=pltpu.CompilerParams(needs_layout_passes=False),
)(_sc_hist_body)


def _dot_body(w_ref, r_ref, out_ref):
    w = (w_ref[0, :].reshape(_W_ROWS, _N_COLS)
         + w_ref[1, :].reshape(_W_ROWS, _N_COLS))
    out_ref[...] = jnp.sum(w * r_ref[...]).reshape(1, 1)


_dot_call = pl.pallas_call(
    _dot_body,
    out_shape=jax.ShapeDtypeStruct((1, 1), jnp.float32),
)


def kernel(one_index, zero_index, target, input):
    one2 = one_index.reshape(_IDX_ROWS, _IDX_BLK)
    zero2 = zero_index.reshape(_IDX_ROWS, _IDX_BLK)
    weights = _sc_hist_call(one2, zero2)
    rowsum = _rowsum_call(input, target)
    loss = _dot_call(weights, rowsum)
    return loss[0, 0]


# speedup vs baseline: 1.5860x; 1.0633x over previous
import functools

import jax
import jax.numpy as jnp
from jax import lax
from jax.experimental import pallas as pl
from jax.experimental.pallas import tpu as pltpu
from jax.experimental.pallas import tpu_sc as plsc

_ALPHA = 0.8
_N_ROWS = 16384
_N_COLS = 128
_N_IDX = 8192

_ROW_BLK = 4096
_GRID = _N_ROWS // _ROW_BLK

_NC = 2
_NS = 16
_NW = _NC * _NS
_LANES = 16
_IDX_BLK = 128
_IDX_ROWS = _N_IDX // _IDX_BLK
_ROWS_PER_TILE = _IDX_ROWS // _NW
_HIST_SLICE = _N_ROWS // _NS


_W_ROWS = _N_ROWS // _N_COLS
_SUB = _ROW_BLK // _N_COLS


def _rowsum_body(inp_ref, tgt_ref, out_ref):
    ones = jnp.ones((1, _N_COLS), jnp.float32)
    for k in range(_SUB):
        d = (inp_ref[pl.ds(k * _N_COLS, _N_COLS), :]
             - tgt_ref[pl.ds(k * _N_COLS, _N_COLS), :])
        out_ref[pl.ds(k, 1), :] = jax.lax.dot_general(
            ones, d * d, (((1,), (1,)), ((), ())),
            preferred_element_type=jnp.float32)


_rowsum_call = pl.pallas_call(
    _rowsum_body,
    grid=(_GRID,),
    in_specs=[
        pl.BlockSpec((_ROW_BLK, _N_COLS), lambda i: (i, 0)),
        pl.BlockSpec((_ROW_BLK, _N_COLS), lambda i: (i, 0)),
    ],
    out_specs=pl.BlockSpec((_SUB, _N_COLS), lambda i: (i, 0)),
    out_shape=jax.ShapeDtypeStruct((_W_ROWS, _N_COLS), jnp.float32),
)


def _sc_hist_body(one_hbm, zero_hbm, w_hbm,
                  idx1_v, idx0_v, vals1_v, vals0_v, zeros_v, hist_s):
    cid = lax.axis_index("c")
    sid = lax.axis_index("s")

    pltpu.sync_copy(one_hbm.at[pl.ds(sid * _ROWS_PER_TILE * _NC
                                     + cid * _ROWS_PER_TILE, _ROWS_PER_TILE)],
                    idx1_v)
    pltpu.sync_copy(zero_hbm.at[pl.ds(sid * _ROWS_PER_TILE * _NC
                                      + cid * _ROWS_PER_TILE, _ROWS_PER_TILE)],
                    idx0_v)

    w1 = jnp.full((_LANES,), 1.0 - _ALPHA, jnp.float32)
    w0 = jnp.full((_LANES,), _ALPHA, jnp.float32)
    zv = jnp.zeros((_LANES,), jnp.float32)
    for i in range(_IDX_BLK // _LANES):
        vals1_v[pl.ds(i * _LANES, _LANES)] = w1
        vals0_v[pl.ds(i * _LANES, _LANES)] = w0
    for i in range(_HIST_SLICE // _LANES):
        zeros_v[pl.ds(i * _LANES, _LANES)] = zv

    pltpu.sync_copy(zeros_v, hist_s.at[pl.ds(sid * _HIST_SLICE, _HIST_SLICE)])
    plsc.subcore_barrier()

    for r in range(_ROWS_PER_TILE):
        pltpu.sync_copy(vals1_v, hist_s.at[idx1_v.at[r]], add=True)
        pltpu.sync_copy(vals0_v, hist_s.at[idx0_v.at[r]], add=True)
    plsc.subcore_barrier()

    pltpu.sync_copy(hist_s.at[pl.ds(sid * _HIST_SLICE, _HIST_SLICE)],
                    w_hbm.at[cid, pl.ds(sid * _HIST_SLICE, _HIST_SLICE)])


_sc_hist_call = functools.partial(
    pl.kernel,
    mesh=plsc.VectorSubcoreMesh(core_axis_name="c", subcore_axis_name="s"),
    out_type=jax.ShapeDtypeStruct((_NC, _N_ROWS), jnp.float32),
    scratch_types=[
        pltpu.VMEM((_ROWS_PER_TILE, _IDX_BLK), jnp.int32),
        pltpu.VMEM((_ROWS_PER_TILE, _IDX_BLK), jnp.int32),
        pltpu.VMEM((_IDX_BLK,), jnp.float32),
        pltpu.VMEM((_IDX_BLK,), jnp.float32),
        pltpu.VMEM((_HIST_SLICE,), jnp.float32),
        pltpu.VMEM_SHARED((_N_ROWS,), jnp.float32),
    ],
    compiler_params=pltpu.CompilerParams(needs_layout_passes=False),
)(_sc_hist_body)


def _dot_body(w_ref, r_ref, out_ref):
    w = (w_ref[0, :].reshape(_W_ROWS, _N_COLS)
         + w_ref[1, :].reshape(_W_ROWS, _N_COLS))
    out_ref[...] = jnp.sum(w * r_ref[...]).reshape(1, 1)


_dot_call = pl.pallas_call(
    _dot_body,
    out_shape=jax.ShapeDtypeStruct((1, 1), jnp.float32),
)


def kernel(one_index, zero_index, target, input):
    one2 = one_index.reshape(_IDX_ROWS, _IDX_BLK)
    zero2 = zero_index.reshape(_IDX_ROWS, _IDX_BLK)
    weights = _sc_hist_call(one2, zero2)
    rowsum = _rowsum_call(input, target)
    loss = _dot_call(weights, rowsum)
    return loss[0, 0]


# speedup vs baseline: 1.6031x; 1.0108x over previous
import functools

import jax
import jax.numpy as jnp
from jax import lax
from jax.experimental import pallas as pl
from jax.experimental.pallas import tpu as pltpu
from jax.experimental.pallas import tpu_sc as plsc

_ALPHA = 0.8
_N_ROWS = 16384
_N_COLS = 128
_N_IDX = 8192

_ROW_BLK = 8192
_GRID = _N_ROWS // _ROW_BLK

_NC = 2
_NS = 16
_NW = _NC * _NS
_LANES = 16
_IDX_BLK = 128
_IDX_ROWS = _N_IDX // _IDX_BLK
_ROWS_PER_TILE = _IDX_ROWS // _NW
_HIST_SLICE = _N_ROWS // _NS


_W_ROWS = _N_ROWS // _N_COLS
_SUB = _ROW_BLK // _N_COLS


def _rowsum_body(inp_ref, tgt_ref, out_ref):
    ones = jnp.ones((1, _N_COLS), jnp.float32)
    for k in range(_SUB):
        d = (inp_ref[pl.ds(k * _N_COLS, _N_COLS), :]
             - tgt_ref[pl.ds(k * _N_COLS, _N_COLS), :])
        out_ref[pl.ds(k, 1), :] = jax.lax.dot_general(
            ones, d * d, (((1,), (1,)), ((), ())),
            preferred_element_type=jnp.float32)


_rowsum_call = pl.pallas_call(
    _rowsum_body,
    grid=(_GRID,),
    in_specs=[
        pl.BlockSpec((_ROW_BLK, _N_COLS), lambda i: (i, 0)),
        pl.BlockSpec((_ROW_BLK, _N_COLS), lambda i: (i, 0)),
    ],
    out_specs=pl.BlockSpec((_SUB, _N_COLS), lambda i: (i, 0)),
    out_shape=jax.ShapeDtypeStruct((_W_ROWS, _N_COLS), jnp.float32),
)


def _sc_hist_body(one_hbm, zero_hbm, w_hbm,
                  idx1_v, idx0_v, vals1_v, vals0_v, zeros_v, hist_s):
    cid = lax.axis_index("c")
    sid = lax.axis_index("s")

    pltpu.sync_copy(one_hbm.at[pl.ds(sid * _ROWS_PER_TILE * _NC
                                     + cid * _ROWS_PER_TILE, _ROWS_PER_TILE)],
                    idx1_v)
    pltpu.sync_copy(zero_hbm.at[pl.ds(sid * _ROWS_PER_TILE * _NC
                                      + cid * _ROWS_PER_TILE, _ROWS_PER_TILE)],
                    idx0_v)

    w1 = jnp.full((_LANES,), 1.0 - _ALPHA, jnp.float32)
    w0 = jnp.full((_LANES,), _ALPHA, jnp.float32)
    zv = jnp.zeros((_LANES,), jnp.float32)
    for i in range(_IDX_BLK // _LANES):
        vals1_v[pl.ds(i * _LANES, _LANES)] = w1
        vals0_v[pl.ds(i * _LANES, _LANES)] = w0
    for i in range(_HIST_SLICE // _LANES):
        zeros_v[pl.ds(i * _LANES, _LANES)] = zv

    pltpu.sync_copy(zeros_v, hist_s.at[pl.ds(sid * _HIST_SLICE, _HIST_SLICE)])
    plsc.subcore_barrier()

    for r in range(_ROWS_PER_TILE):
        pltpu.sync_copy(vals1_v, hist_s.at[idx1_v.at[r]], add=True)
        pltpu.sync_copy(vals0_v, hist_s.at[idx0_v.at[r]], add=True)
    plsc.subcore_barrier()

    pltpu.sync_copy(hist_s.at[pl.ds(sid * _HIST_SLICE, _HIST_SLICE)],
                    w_hbm.at[cid, pl.ds(sid * _HIST_SLICE, _HIST_SLICE)])


_sc_hist_call = functools.partial(
    pl.kernel,
    mesh=plsc.VectorSubcoreMesh(core_axis_name="c", subcore_axis_name="s"),
    out_type=jax.ShapeDtypeStruct((_NC, _N_ROWS), jnp.float32),
    scratch_types=[
        pltpu.VMEM((_ROWS_PER_TILE, _IDX_BLK), jnp.int32),
        pltpu.VMEM((_ROWS_PER_TILE, _IDX_BLK), jnp.int32),
        pltpu.VMEM((_IDX_BLK,), jnp.float32),
        pltpu.VMEM((_IDX_BLK,), jnp.float32),
        pltpu.VMEM((_HIST_SLICE,), jnp.float32),
        pltpu.VMEM_SHARED((_N_ROWS,), jnp.float32),
    ],
    compiler_params=pltpu.CompilerParams(needs_layout_passes=False),
)(_sc_hist_body)


def _dot_body(w_ref, r_ref, out_ref):
    w = (w_ref[0, :].reshape(_W_ROWS, _N_COLS)
         + w_ref[1, :].reshape(_W_ROWS, _N_COLS))
    out_ref[...] = jnp.sum(w * r_ref[...]).reshape(1, 1)


_dot_call = pl.pallas_call(
    _dot_body,
    out_shape=jax.ShapeDtypeStruct((1, 1), jnp.float32),
)


def kernel(one_index, zero_index, target, input):
    one2 = one_index.reshape(_IDX_ROWS, _IDX_BLK)
    zero2 = zero_index.reshape(_IDX_ROWS, _IDX_BLK)
    weights = _sc_hist_call(one2, zero2)
    rowsum = _rowsum_call(input, target)
    loss = _dot_call(weights, rowsum)
    return loss[0, 0]
